# baseline (speedup 1.0000x reference)
import functools
import numpy as np
import jax
import jax.numpy as jnp
from jax.experimental import pallas as pl
from jax.experimental.pallas import tpu as pltpu

_BN = 1.0 / np.sqrt(1.0 + 1e-5)


def _fps_body(x_ref, o_ref, *, npoint):
    x = x_ref[...]
    B, _, N = x.shape
    iota = jax.lax.broadcasted_iota(jnp.int32, (B, N), 1).astype(jnp.float32)

    def step(i, carry):
        dists, far = carry
        oh = (iota == far).astype(x.dtype)
        ct = jnp.sum(x * oh[:, None, :], axis=2)
        dx = x[:, 0, :] - ct[:, 0:1]
        dy = x[:, 1, :] - ct[:, 1:2]
        dz = x[:, 2, :] - ct[:, 2:3]
        d = (dx * dx + dy * dy) + dz * dz
        dists = jnp.minimum(dists, d)
        m = jnp.max(dists, axis=1, keepdims=True)
        nxt = jnp.min(jnp.where(dists == m, iota, float(N)), axis=1,
                      keepdims=True)
        o_ref[pl.ds(i, 1), :, :] = ct[None]
        return dists, nxt

    jax.lax.fori_loop(
        0, npoint, step,
        (jnp.full((B, N), 1e10, x.dtype), jnp.zeros((B, 1), x.dtype)))


def _fps_call(xpay, npoint):
    B = xpay.shape[0]
    out = pl.pallas_call(
        functools.partial(_fps_body, npoint=npoint),
        out_shape=jax.ShapeDtypeStruct((npoint, B, 4), jnp.float32),
    )(xpay)
    return jnp.transpose(out, (1, 2, 0))


def _bqg_body(tab_ref, xt_ref, q_ref, w1_ref, w2_ref, w3_ref, o_ref,
              *, r2, inv_r, ns, C, ch):
    tab = tab_ref[0]
    xt = xt_ref[0]
    q = q_ref[0]
    R = q.shape[0]
    N = xt.shape[1]
    x2 = (xt[0:1, :] * xt[0:1, :] + xt[1:2, :] * xt[1:2, :]
          + xt[2:3, :] * xt[2:3, :])
    q2 = (q[:, 0:1] * q[:, 0:1] + q[:, 1:2] * q[:, 1:2]
          + q[:, 2:3] * q[:, 2:3])
    qx = jnp.dot(q.astype(jnp.bfloat16), xt.astype(jnp.bfloat16),
                 preferred_element_type=jnp.float32)
    sqd = jnp.maximum(q2 - 2.0 * qx + x2, 0.0)
    mask = (sqd < r2).astype(jnp.float32)
    iota = jax.lax.broadcasted_iota(jnp.int32, (R, ch), 1).astype(jnp.float32)
    shift = jnp.concatenate(
        [jnp.zeros((R, C), jnp.float32), q * inv_r], axis=1)
    Cout = w3_ref.shape[1]

    def extract_mlp(oh, tab_c):
        v = jnp.dot(oh, tab_c, preferred_element_type=jnp.float32,
                    precision=jax.lax.Precision.HIGHEST) - shift
        v = jax.nn.relu(jnp.dot(v, w1_ref[...],
                                preferred_element_type=jnp.float32) * _BN)
        v = jax.nn.relu(jnp.dot(v, w2_ref[...],
                                preferred_element_type=jnp.float32) * _BN)
        return jax.nn.relu(jnp.dot(v, w3_ref[...],
                                   preferred_element_type=jnp.float32) * _BN)

    acc = jnp.full((R, Cout), -1e30, jnp.float32)
    cnt = jnp.zeros((R, 1), jnp.float32)
    for c in range(N // ch):
        m0 = mask[:, c * ch:(c + 1) * ch]
        tab_c = tab[c * ch:(c + 1) * ch, :]
        limit_c = jnp.max(jnp.sum(m0, axis=1)).astype(jnp.int32)

        def cstep(j, carry, tab_c=tab_c):
            m, cnt, acc = carry
            cur = jnp.min(jnp.where(m > 0.0, iota, float(ch)), axis=1,
                          keepdims=True)
            ok = jnp.logical_and(cur < ch, cnt < float(ns))
            oh = (iota == cur).astype(jnp.float32)
            val = extract_mlp(oh, tab_c)
            acc = jnp.where(ok, jnp.maximum(acc, val), acc)
            cnt = cnt + ok.astype(jnp.float32)
            return m - oh, cnt, acc

        _, cnt, acc = jax.lax.fori_loop(0, limit_c, cstep, (m0, cnt, acc))

    oh_last = (iota == float(ch - 1)).astype(jnp.float32)
    padv = extract_mlp(oh_last, tab[N - ch:N, :])
    o_ref[0] = jnp.where(cnt > 0.0, acc, padv)


def _bqg_call(table, xyz_t, q, w1t, w2t, w3t, radius, ns, R):
    B, N, C3 = table.shape
    nq = q.shape[1]
    Cout = w3t.shape[1]
    full = lambda w: pl.BlockSpec(w.shape, lambda b, i: (0, 0))
    return pl.pallas_call(
        functools.partial(_bqg_body, r2=radius * radius, inv_r=1.0 / radius,
                          ns=ns, C=C3 - 3, ch=min(256, N)),
        grid=(B, nq // R),
        in_specs=[
            pl.BlockSpec((1, N, C3), lambda b, i: (b, 0, 0)),
            pl.BlockSpec((1, 3, N), lambda b, i: (b, 0, 0)),
            pl.BlockSpec((1, R, 3), lambda b, i: (b, i, 0)),
            full(w1t), full(w2t), full(w3t),
        ],
        out_specs=pl.BlockSpec((1, R, Cout), lambda b, i: (b, i, 0)),
        out_shape=jax.ShapeDtypeStruct((B, nq, Cout), jnp.float32),
    )(table, xyz_t, q, w1t, w2t, w3t)


def _proj_body(f_ref, fc1_ref, b1_ref, wq_ref, wk_ref, wv_ref,
               q_ref, k_ref, v_ref):
    f = f_ref[0]
    x = jnp.dot(f, fc1_ref[...], preferred_element_type=jnp.float32) + b1_ref[...]
    q_ref[0] = jnp.dot(x, wq_ref[...], preferred_element_type=jnp.float32)
    k_ref[0] = jnp.dot(x, wk_ref[...], preferred_element_type=jnp.float32)
    v_ref[0] = jnp.dot(x, wv_ref[...], preferred_element_type=jnp.float32)


def _proj_call(f, fc1t, b1, wqt, wkt, wvt):
    B, N, _ = f.shape
    d = wqt.shape[1]
    full = lambda w: pl.BlockSpec(w.shape, lambda b: (0, 0))
    out = jax.ShapeDtypeStruct((B, N, d), jnp.float32)
    return pl.pallas_call(
        _proj_body,
        grid=(B,),
        in_specs=[pl.BlockSpec((1, N, f.shape[2]), lambda b: (b, 0, 0)),
                  full(fc1t), full(b1), full(wqt), full(wkt), full(wvt)],
        out_specs=[pl.BlockSpec((1, N, d), lambda b: (b, 0, 0))] * 3,
        out_shape=[out, out, out],
    )(f, fc1t, b1, wqt, wkt, wvt)


def _attn_body(qxyz_ref, xt_ref, tab_ref, qp_ref, pre_ref,
               d1_ref, d1b_ref, d2_ref, d2b_ref,
               g1_ref, g1b_ref, g2_ref, g2b_ref,
               fc2_ref, fc2b_ref, o_ref, scr, *, K, d):
    qxyz = qxyz_ref[0]
    xt = xt_ref[0]
    tab = tab_ref[0]
    R = qxyz.shape[0]
    N = xt.shape[1]
    x2 = (xt[0:1, :] * xt[0:1, :] + xt[1:2, :] * xt[1:2, :]
          + xt[2:3, :] * xt[2:3, :])
    q2 = (qxyz[:, 0:1] * qxyz[:, 0:1] + qxyz[:, 1:2] * qxyz[:, 1:2]
          + qxyz[:, 2:3] * qxyz[:, 2:3])
    qx = jnp.dot(qxyz.astype(jnp.bfloat16), xt.astype(jnp.bfloat16),
                 preferred_element_type=jnp.float32)
    sqd = jnp.maximum(q2 - 2.0 * qx + x2, 0.0)
    iota = jax.lax.broadcasted_iota(jnp.int32, (R, N), 1).astype(jnp.float32)

    def step(j, dmat):
        mval = jnp.min(dmat, axis=1, keepdims=True)
        cur = jnp.min(jnp.where(dmat == mval, iota, float(N)), axis=1,
                      keepdims=True)
        oh = (iota == cur).astype(jnp.float32)
        val_kv = jnp.dot(oh, tab[:, :2 * d], preferred_element_type=jnp.float32)
        val_x = jnp.dot(oh, tab[:, 2 * d:], preferred_element_type=jnp.float32,
                        precision=jax.lax.Precision.HIGHEST)
        scr[pl.ds(j, 1), :, :2 * d] = val_kv[None]
        scr[pl.ds(j, 1), :, 2 * d:] = val_x[None]
        return dmat + oh * jnp.float32(1e30)

    jax.lax.fori_loop(0, K, step, sqd)

    gat = scr[...]
    kk = gat[:, :, :d]
    vv = gat[:, :, d:2 * d]
    nxyz = gat[:, :, 2 * d:]
    delta = (qxyz[None, :, :] - nxyz).reshape(K * R, 3)
    pe = jax.nn.relu(
        jnp.dot(delta, d1_ref[...], preferred_element_type=jnp.float32)
        + d1b_ref[...])
    pe = jnp.dot(pe, d2_ref[...], preferred_element_type=jnp.float32) + d2b_ref[...]
    pe3 = pe.reshape(K, R, d)
    qp = qp_ref[0]
    g = (qp[None, :, :] - kk + pe3).reshape(K * R, d)
    a = jax.nn.relu(
        jnp.dot(g, g1_ref[...], preferred_element_type=jnp.float32)
        + g1b_ref[...])
    a = jnp.dot(a, g2_ref[...], preferred_element_type=jnp.float32) + g2b_ref[...]
    a = a.reshape(K, R, d) / np.sqrt(float(d)).astype(np.float32)
    m = jnp.max(a, axis=0, keepdims=True)
    e = jnp.exp(a - m)
    a = e / jnp.sum(e, axis=0, keepdims=True)
    res = jnp.sum(a * (vv + pe3), axis=0)
    o_ref[0] = (jnp.dot(res, fc2_ref[...], preferred_element_type=jnp.float32)
                + fc2b_ref[...] + pre_ref[0])


def _attn_call(qxyz, xyz_t, tab, qp, pre, p, K, R):
    B, N, _ = qxyz.shape
    d = qp.shape[2]
    dp = pre.shape[2]
    full = lambda w: pl.BlockSpec(w.shape, lambda b, i: (0, 0))
    d1t = p['d1_w'].T
    d2t = p['d2_w'].T
    g1t = p['g1_w'].T
    g2t = p['g2_w'].T
    fc2t = p['fc2_w'].T
    row = lambda v: v[None, :]
    return pl.pallas_call(
        functools.partial(_attn_body, K=K, d=d),
        grid=(B, N // R),
        in_specs=[
            pl.BlockSpec((1, R, 3), lambda b, i: (b, i, 0)),
            pl.BlockSpec((1, 3, N), lambda b, i: (b, 0, 0)),
            pl.BlockSpec((1, N, 2 * d + 3), lambda b, i: (b, 0, 0)),
            pl.BlockSpec((1, R, d), lambda b, i: (b, i, 0)),
            pl.BlockSpec((1, R, dp), lambda b, i: (b, i, 0)),
            full(d1t), full(row(p['d1_b'])), full(d2t), full(row(p['d2_b'])),
            full(g1t), full(row(p['g1_b'])), full(g2t), full(row(p['g2_b'])),
            full(fc2t), full(row(p['fc2_b'])),
        ],
        out_specs=pl.BlockSpec((1, R, dp), lambda b, i: (b, i, 0)),
        out_shape=jax.ShapeDtypeStruct((B, N, dp), jnp.float32),
        scratch_shapes=[pltpu.VMEM((K, R, 2 * d + 3), jnp.float32)],
    )(qxyz, xyz_t, tab, qp, pre, d1t, row(p['d1_b']), d2t, row(p['d2_b']),
      g1t, row(p['g1_b']), g2t, row(p['g2_b']), fc2t, row(p['fc2_b']))


def _sa_weights(wlist):
    w1 = jnp.concatenate([wlist[0][:, 3:], wlist[0][:, :3]], axis=1)
    return w1.T, wlist[1].T, wlist[2].T


def _transformer(xyz, xyz_t, f, p, K, R):
    q, k, v = _proj_call(f, p['fc1_w'].T, p['fc1_b'][None, :],
                         p['wq'].T, p['wk'].T, p['wv'].T)
    tab = jnp.concatenate([k, v, xyz], axis=-1)
    return _attn_call(xyz, xyz_t, tab, q, f, p, K, R)


def _forward(pointcloud, params, cfg):
    pc = pointcloud.astype(jnp.float32)
    B, N, _ = pc.shape
    xyz = pc[..., :3]
    feats = pc[..., 3:]
    xyz_t = jnp.transpose(xyz, (0, 2, 1))

    np1, r1, ns1 = cfg['sa1']
    np2, r2, ns2 = cfg['sa2']

    pay1 = jnp.broadcast_to(
        jax.lax.iota(jnp.float32, N)[None, None, :], (B, 1, N))
    fps1 = _fps_call(jnp.concatenate([xyz_t, pay1], axis=1), np1)
    xyz1_t = fps1[:, 0:3, :]
    inds1f = fps1[:, 3:4, :]
    xyz1 = jnp.transpose(xyz1_t, (0, 2, 1))

    tab1 = jnp.concatenate([feats, xyz * (1.0 / r1)], axis=-1)
    f1 = _bqg_call(tab1, xyz_t, xyz1, *_sa_weights(params['sa1']),
                   r1, ns1, cfg['R_bq1'])
    f1 = _transformer(xyz1, xyz1_t, f1, params['t1'], cfg['k'], cfg['R_at1'])

    fps2 = _fps_call(jnp.concatenate([xyz1_t, inds1f], axis=1), np2)
    xyz2_t = fps2[:, 0:3, :]
    fp2f = fps2[:, 3, :]
    xyz2 = jnp.transpose(xyz2_t, (0, 2, 1))

    tab2 = jnp.concatenate([f1, xyz1 * (1.0 / r2)], axis=-1)
    f2 = _bqg_call(tab2, xyz1_t, xyz2, *_sa_weights(params['sa2']),
                   r2, ns2, cfg['R_bq2'])
    f2 = _transformer(xyz2, xyz2_t, f2, params['t2'], cfg['k'], cfg['R_at2'])

    return (jnp.transpose(f2, (0, 2, 1)), xyz2, fp2f.astype(jnp.int32))


_CFG = {
    'sa1': (2048, 0.04, 64),
    'sa2': (1024, 0.1, 32),
    'k': 16,
    'R_bq1': 128,
    'R_bq2': 128,
    'R_at1': 128,
    'R_at2': 128,
}


def kernel(pointcloud, params):
    return _forward(pointcloud, params, _CFG)


# speedup vs baseline: 1.0710x; 1.0710x over previous
import functools
import numpy as np
import jax
import jax.numpy as jnp
from jax.experimental import pallas as pl
from jax.experimental.pallas import tpu as pltpu

_BN = 1.0 / np.sqrt(1.0 + 1e-5)


def _fps_body(x_ref, o_ref, *, npoint):
    x = x_ref[...]
    B, _, N = x.shape
    iota = jax.lax.broadcasted_iota(jnp.int32, (B, N), 1).astype(jnp.float32)

    def step(i, carry):
        dists, far = carry
        oh = (iota == far).astype(x.dtype)
        ct = jnp.sum(x * oh[:, None, :], axis=2)
        dx = x[:, 0, :] - ct[:, 0:1]
        dy = x[:, 1, :] - ct[:, 1:2]
        dz = x[:, 2, :] - ct[:, 2:3]
        d = (dx * dx + dy * dy) + dz * dz
        dists = jnp.minimum(dists, d)
        m = jnp.max(dists, axis=1, keepdims=True)
        nxt = jnp.min(jnp.where(dists == m, iota, float(N)), axis=1,
                      keepdims=True)
        o_ref[pl.ds(i, 1), :, :] = ct[None]
        return dists, nxt

    jax.lax.fori_loop(
        0, npoint, step,
        (jnp.full((B, N), 1e10, x.dtype), jnp.zeros((B, 1), x.dtype)))


def _fps_call(xpay, npoint):
    B = xpay.shape[0]
    out = pl.pallas_call(
        functools.partial(_fps_body, npoint=npoint),
        out_shape=jax.ShapeDtypeStruct((npoint, B, 4), jnp.float32),
    )(xpay)
    return jnp.transpose(out, (1, 2, 0))


def _bqg_body(tab_ref, xt_ref, q_ref, w1_ref, w2_ref, w3_ref, o_ref,
              *, r2, inv_r, ns, C, ch):
    tab = tab_ref[0]
    xt = xt_ref[0]
    q = q_ref[0]
    R = q.shape[0]
    N = xt.shape[1]
    x2 = (xt[0:1, :] * xt[0:1, :] + xt[1:2, :] * xt[1:2, :]
          + xt[2:3, :] * xt[2:3, :])
    q2 = (q[:, 0:1] * q[:, 0:1] + q[:, 1:2] * q[:, 1:2]
          + q[:, 2:3] * q[:, 2:3])
    qx = jnp.dot(q.astype(jnp.bfloat16), xt.astype(jnp.bfloat16),
                 preferred_element_type=jnp.float32)
    sqd = jnp.maximum(q2 - 2.0 * qx + x2, 0.0)
    mask = (sqd < r2).astype(jnp.float32)
    iota = jax.lax.broadcasted_iota(jnp.int32, (R, ch), 1).astype(jnp.float32)
    shift = jnp.concatenate(
        [jnp.zeros((R, C), jnp.float32), q * inv_r], axis=1)
    Cout = w3_ref.shape[1]

    def extract_mlp(oh, tab_c):
        v = jnp.dot(oh, tab_c, preferred_element_type=jnp.float32,
                    precision=jax.lax.Precision.HIGHEST) - shift
        v = jax.nn.relu(jnp.dot(v, w1_ref[...],
                                preferred_element_type=jnp.float32) * _BN)
        v = jax.nn.relu(jnp.dot(v, w2_ref[...],
                                preferred_element_type=jnp.float32) * _BN)
        return jax.nn.relu(jnp.dot(v, w3_ref[...],
                                   preferred_element_type=jnp.float32) * _BN)

    acc = jnp.full((R, Cout), -1e30, jnp.float32)
    cnt = jnp.zeros((R, 1), jnp.float32)
    for c in range(N // ch):
        m0 = mask[:, c * ch:(c + 1) * ch]
        tab_c = tab[c * ch:(c + 1) * ch, :]
        limit_c = jnp.max(jnp.sum(m0, axis=1)).astype(jnp.int32)

        def cstep(j, carry, tab_c=tab_c):
            m, cnt, acc = carry
            cur = jnp.min(jnp.where(m > 0.0, iota, float(ch)), axis=1,
                          keepdims=True)
            ok = jnp.logical_and(cur < ch, cnt < float(ns))
            oh = (iota == cur).astype(jnp.float32)
            val = extract_mlp(oh, tab_c)
            acc = jnp.where(ok, jnp.maximum(acc, val), acc)
            cnt = cnt + ok.astype(jnp.float32)
            return m - oh, cnt, acc

        _, cnt, acc = jax.lax.fori_loop(0, limit_c, cstep, (m0, cnt, acc))

    oh_last = (iota == float(ch - 1)).astype(jnp.float32)
    padv = extract_mlp(oh_last, tab[N - ch:N, :])
    o_ref[0] = jnp.where(cnt > 0.0, acc, padv)


def _bqg_call(table, xyz_t, q, w1t, w2t, w3t, radius, ns, R):
    B, N, C3 = table.shape
    nq = q.shape[1]
    Cout = w3t.shape[1]
    full = lambda w: pl.BlockSpec(w.shape, lambda b, i: (0, 0))
    return pl.pallas_call(
        functools.partial(_bqg_body, r2=radius * radius, inv_r=1.0 / radius,
                          ns=ns, C=C3 - 3, ch=min(512, N)),
        grid=(B, nq // R),
        in_specs=[
            pl.BlockSpec((1, N, C3), lambda b, i: (b, 0, 0)),
            pl.BlockSpec((1, 3, N), lambda b, i: (b, 0, 0)),
            pl.BlockSpec((1, R, 3), lambda b, i: (b, i, 0)),
            full(w1t), full(w2t), full(w3t),
        ],
        out_specs=pl.BlockSpec((1, R, Cout), lambda b, i: (b, i, 0)),
        out_shape=jax.ShapeDtypeStruct((B, nq, Cout), jnp.float32),
    )(table, xyz_t, q, w1t, w2t, w3t)


def _proj_body(f_ref, fc1_ref, b1_ref, wq_ref, wk_ref, wv_ref,
               q_ref, k_ref, v_ref):
    f = f_ref[0]
    x = jnp.dot(f, fc1_ref[...], preferred_element_type=jnp.float32) + b1_ref[...]
    q_ref[0] = jnp.dot(x, wq_ref[...], preferred_element_type=jnp.float32)
    k_ref[0] = jnp.dot(x, wk_ref[...], preferred_element_type=jnp.float32)
    v_ref[0] = jnp.dot(x, wv_ref[...], preferred_element_type=jnp.float32)


def _proj_call(f, fc1t, b1, wqt, wkt, wvt):
    B, N, _ = f.shape
    d = wqt.shape[1]
    full = lambda w: pl.BlockSpec(w.shape, lambda b: (0, 0))
    out = jax.ShapeDtypeStruct((B, N, d), jnp.float32)
    return pl.pallas_call(
        _proj_body,
        grid=(B,),
        in_specs=[pl.BlockSpec((1, N, f.shape[2]), lambda b: (b, 0, 0)),
                  full(fc1t), full(b1), full(wqt), full(wkt), full(wvt)],
        out_specs=[pl.BlockSpec((1, N, d), lambda b: (b, 0, 0))] * 3,
        out_shape=[out, out, out],
    )(f, fc1t, b1, wqt, wkt, wvt)


def _attn_body(qxyz_ref, xt_ref, tab_ref, qp_ref, pre_ref,
               d1_ref, d1b_ref, d2_ref, d2b_ref,
               g1_ref, g1b_ref, g2_ref, g2b_ref,
               fc2_ref, fc2b_ref, o_ref, scr, *, K, d):
    qxyz = qxyz_ref[0]
    xt = xt_ref[0]
    tab = tab_ref[0]
    R = qxyz.shape[0]
    N = xt.shape[1]
    x2 = (xt[0:1, :] * xt[0:1, :] + xt[1:2, :] * xt[1:2, :]
          + xt[2:3, :] * xt[2:3, :])
    q2 = (qxyz[:, 0:1] * qxyz[:, 0:1] + qxyz[:, 1:2] * qxyz[:, 1:2]
          + qxyz[:, 2:3] * qxyz[:, 2:3])
    qx = jnp.dot(qxyz.astype(jnp.bfloat16), xt.astype(jnp.bfloat16),
                 preferred_element_type=jnp.float32)
    sqd = jnp.maximum(q2 - 2.0 * qx + x2, 0.0)
    iota = jax.lax.broadcasted_iota(jnp.int32, (R, N), 1).astype(jnp.float32)

    def step(j, dmat):
        mval = jnp.min(dmat, axis=1, keepdims=True)
        cur = jnp.min(jnp.where(dmat == mval, iota, float(N)), axis=1,
                      keepdims=True)
        oh = (iota == cur).astype(jnp.float32)
        val_kv = jnp.dot(oh, tab[:, :2 * d], preferred_element_type=jnp.float32)
        val_x = jnp.dot(oh, tab[:, 2 * d:], preferred_element_type=jnp.float32,
                        precision=jax.lax.Precision.HIGHEST)
        scr[pl.ds(j, 1), :, :2 * d] = val_kv[None]
        scr[pl.ds(j, 1), :, 2 * d:] = val_x[None]
        return dmat + oh * jnp.float32(1e30)

    jax.lax.fori_loop(0, K, step, sqd)

    gat = scr[...]
    kk = gat[:, :, :d]
    vv = gat[:, :, d:2 * d]
    nxyz = gat[:, :, 2 * d:]
    delta = (qxyz[None, :, :] - nxyz).reshape(K * R, 3)
    pe = jax.nn.relu(
        jnp.dot(delta, d1_ref[...], preferred_element_type=jnp.float32)
        + d1b_ref[...])
    pe = jnp.dot(pe, d2_ref[...], preferred_element_type=jnp.float32) + d2b_ref[...]
    pe3 = pe.reshape(K, R, d)
    qp = qp_ref[0]
    g = (qp[None, :, :] - kk + pe3).reshape(K * R, d)
    a = jax.nn.relu(
        jnp.dot(g, g1_ref[...], preferred_element_type=jnp.float32)
        + g1b_ref[...])
    a = jnp.dot(a, g2_ref[...], preferred_element_type=jnp.float32) + g2b_ref[...]
    a = a.reshape(K, R, d) / np.sqrt(float(d)).astype(np.float32)
    m = jnp.max(a, axis=0, keepdims=True)
    e = jnp.exp(a - m)
    a = e / jnp.sum(e, axis=0, keepdims=True)
    res = jnp.sum(a * (vv + pe3), axis=0)
    o_ref[0] = (jnp.dot(res, fc2_ref[...], preferred_element_type=jnp.float32)
                + fc2b_ref[...] + pre_ref[0])


def _attn_call(qxyz, xyz_t, tab, qp, pre, p, K, R):
    B, N, _ = qxyz.shape
    d = qp.shape[2]
    dp = pre.shape[2]
    full = lambda w: pl.BlockSpec(w.shape, lambda b, i: (0, 0))
    d1t = p['d1_w'].T
    d2t = p['d2_w'].T
    g1t = p['g1_w'].T
    g2t = p['g2_w'].T
    fc2t = p['fc2_w'].T
    row = lambda v: v[None, :]
    return pl.pallas_call(
        functools.partial(_attn_body, K=K, d=d),
        grid=(B, N // R),
        in_specs=[
            pl.BlockSpec((1, R, 3), lambda b, i: (b, i, 0)),
            pl.BlockSpec((1, 3, N), lambda b, i: (b, 0, 0)),
            pl.BlockSpec((1, N, 2 * d + 3), lambda b, i: (b, 0, 0)),
            pl.BlockSpec((1, R, d), lambda b, i: (b, i, 0)),
            pl.BlockSpec((1, R, dp), lambda b, i: (b, i, 0)),
            full(d1t), full(row(p['d1_b'])), full(d2t), full(row(p['d2_b'])),
            full(g1t), full(row(p['g1_b'])), full(g2t), full(row(p['g2_b'])),
            full(fc2t), full(row(p['fc2_b'])),
        ],
        out_specs=pl.BlockSpec((1, R, dp), lambda b, i: (b, i, 0)),
        out_shape=jax.ShapeDtypeStruct((B, N, dp), jnp.float32),
        scratch_shapes=[pltpu.VMEM((K, R, 2 * d + 3), jnp.float32)],
    )(qxyz, xyz_t, tab, qp, pre, d1t, row(p['d1_b']), d2t, row(p['d2_b']),
      g1t, row(p['g1_b']), g2t, row(p['g2_b']), fc2t, row(p['fc2_b']))


def _sa_weights(wlist):
    w1 = jnp.concatenate([wlist[0][:, 3:], wlist[0][:, :3]], axis=1)
    return w1.T, wlist[1].T, wlist[2].T


def _transformer(xyz, xyz_t, f, p, K, R):
    q, k, v = _proj_call(f, p['fc1_w'].T, p['fc1_b'][None, :],
                         p['wq'].T, p['wk'].T, p['wv'].T)
    tab = jnp.concatenate([k, v, xyz], axis=-1)
    return _attn_call(xyz, xyz_t, tab, q, f, p, K, R)


def _forward(pointcloud, params, cfg):
    pc = pointcloud.astype(jnp.float32)
    B, N, _ = pc.shape
    xyz = pc[..., :3]
    feats = pc[..., 3:]
    xyz_t = jnp.transpose(xyz, (0, 2, 1))

    np1, r1, ns1 = cfg['sa1']
    np2, r2, ns2 = cfg['sa2']

    pay1 = jnp.broadcast_to(
        jax.lax.iota(jnp.float32, N)[None, None, :], (B, 1, N))
    fps1 = _fps_call(jnp.concatenate([xyz_t, pay1], axis=1), np1)
    xyz1_t = fps1[:, 0:3, :]
    inds1f = fps1[:, 3:4, :]
    xyz1 = jnp.transpose(xyz1_t, (0, 2, 1))

    tab1 = jnp.concatenate([feats, xyz * (1.0 / r1)], axis=-1)
    f1 = _bqg_call(tab1, xyz_t, xyz1, *_sa_weights(params['sa1']),
                   r1, ns1, cfg['R_bq1'])
    f1 = _transformer(xyz1, xyz1_t, f1, params['t1'], cfg['k'], cfg['R_at1'])

    fps2 = _fps_call(jnp.concatenate([xyz1_t, inds1f], axis=1), np2)
    xyz2_t = fps2[:, 0:3, :]
    fp2f = fps2[:, 3, :]
    xyz2 = jnp.transpose(xyz2_t, (0, 2, 1))

    tab2 = jnp.concatenate([f1, xyz1 * (1.0 / r2)], axis=-1)
    f2 = _bqg_call(tab2, xyz1_t, xyz2, *_sa_weights(params['sa2']),
                   r2, ns2, cfg['R_bq2'])
    f2 = _transformer(xyz2, xyz2_t, f2, params['t2'], cfg['k'], cfg['R_at2'])

    return (jnp.transpose(f2, (0, 2, 1)), xyz2, fp2f.astype(jnp.int32))


_CFG = {
    'sa1': (2048, 0.04, 64),
    'sa2': (1024, 0.1, 32),
    'k': 16,
    'R_bq1': 128,
    'R_bq2': 128,
    'R_at1': 128,
    'R_at2': 128,
}


def kernel(pointcloud, params):
    return _forward(pointcloud, params, _CFG)


# speedup vs baseline: 1.1694x; 1.0919x over previous
import functools
import numpy as np
import jax
import jax.numpy as jnp
from jax.experimental import pallas as pl
from jax.experimental.pallas import tpu as pltpu

_BN = 1.0 / np.sqrt(1.0 + 1e-5)


def _fps_body(x_ref, o_ref, *, npoint):
    x = x_ref[...]
    B, _, N = x.shape
    iota = jax.lax.broadcasted_iota(jnp.int32, (B, N), 1).astype(jnp.float32)

    def step(i, carry):
        dists, far = carry
        oh = (iota == far).astype(x.dtype)
        ct = jnp.sum(x * oh[:, None, :], axis=2)
        dx = x[:, 0, :] - ct[:, 0:1]
        dy = x[:, 1, :] - ct[:, 1:2]
        dz = x[:, 2, :] - ct[:, 2:3]
        d = (dx * dx + dy * dy) + dz * dz
        dists = jnp.minimum(dists, d)
        m = jnp.max(dists, axis=1, keepdims=True)
        nxt = jnp.min(jnp.where(dists == m, iota, float(N)), axis=1,
                      keepdims=True)
        o_ref[pl.ds(i, 1), :, :] = ct[None]
        return dists, nxt

    jax.lax.fori_loop(
        0, npoint, step,
        (jnp.full((B, N), 1e10, x.dtype), jnp.zeros((B, 1), x.dtype)))


def _fps_call(xpay, npoint):
    B = xpay.shape[0]
    out = pl.pallas_call(
        functools.partial(_fps_body, npoint=npoint),
        out_shape=jax.ShapeDtypeStruct((npoint, B, 4), jnp.float32),
    )(xpay)
    return jnp.transpose(out, (1, 2, 0))


def _bqg_body(tab_ref, xt_ref, q_ref, w1_ref, w2_ref, w3_ref, o_ref,
              *, r2, inv_r, ns, C, ch):
    tab = tab_ref[0]
    xt = xt_ref[0]
    q = q_ref[0]
    R = q.shape[0]
    N = xt.shape[1]
    x2 = (xt[0:1, :] * xt[0:1, :] + xt[1:2, :] * xt[1:2, :]
          + xt[2:3, :] * xt[2:3, :])
    q2 = (q[:, 0:1] * q[:, 0:1] + q[:, 1:2] * q[:, 1:2]
          + q[:, 2:3] * q[:, 2:3])
    qx = jnp.dot(q.astype(jnp.bfloat16), xt.astype(jnp.bfloat16),
                 preferred_element_type=jnp.float32)
    sqd = jnp.maximum(q2 - 2.0 * qx + x2, 0.0)
    mask = (sqd < r2).astype(jnp.float32)
    iota = jax.lax.broadcasted_iota(jnp.int32, (R, ch), 1).astype(jnp.float32)
    shift = jnp.concatenate(
        [jnp.zeros((R, C), jnp.float32), q * inv_r], axis=1)
    Cout = w3_ref.shape[1]

    def extract_mlp(oh, tab_c):
        v = jnp.dot(oh, tab_c, preferred_element_type=jnp.float32,
                    precision=jax.lax.Precision.HIGHEST) - shift
        v = jax.nn.relu(jnp.dot(v, w1_ref[...],
                                preferred_element_type=jnp.float32) * _BN)
        v = jax.nn.relu(jnp.dot(v, w2_ref[...],
                                preferred_element_type=jnp.float32) * _BN)
        return jax.nn.relu(jnp.dot(v, w3_ref[...],
                                   preferred_element_type=jnp.float32) * _BN)

    acc = jnp.full((R, Cout), -1e30, jnp.float32)
    cnt = jnp.zeros((R, 1), jnp.float32)
    for c in range(N // ch):
        m0 = mask[:, c * ch:(c + 1) * ch]
        tab_c = tab[c * ch:(c + 1) * ch, :]
        limit_c = jnp.max(jnp.sum(m0, axis=1)).astype(jnp.int32)

        def cstep(j, carry, tab_c=tab_c):
            m, cnt, acc = carry
            cur = jnp.min(jnp.where(m > 0.0, iota, float(ch)), axis=1,
                          keepdims=True)
            ok = jnp.logical_and(cur < ch, cnt < float(ns))
            oh = (iota == cur).astype(jnp.float32)
            val = extract_mlp(oh, tab_c)
            acc = jnp.where(ok, jnp.maximum(acc, val), acc)
            cnt = cnt + ok.astype(jnp.float32)
            return m - oh, cnt, acc

        _, cnt, acc = jax.lax.fori_loop(0, limit_c, cstep, (m0, cnt, acc))

    oh_last = (iota == float(ch - 1)).astype(jnp.float32)
    padv = extract_mlp(oh_last, tab[N - ch:N, :])
    o_ref[0] = jnp.where(cnt > 0.0, acc, padv)


def _bqg_call(table, xyz_t, q, w1t, w2t, w3t, radius, ns, R):
    B, N, C3 = table.shape
    nq = q.shape[1]
    Cout = w3t.shape[1]
    full = lambda w: pl.BlockSpec(w.shape, lambda b, i: (0, 0))
    return pl.pallas_call(
        functools.partial(_bqg_body, r2=radius * radius, inv_r=1.0 / radius,
                          ns=ns, C=C3 - 3, ch=min(512, N)),
        grid=(B, nq // R),
        in_specs=[
            pl.BlockSpec((1, N, C3), lambda b, i: (b, 0, 0)),
            pl.BlockSpec((1, 3, N), lambda b, i: (b, 0, 0)),
            pl.BlockSpec((1, R, 3), lambda b, i: (b, i, 0)),
            full(w1t), full(w2t), full(w3t),
        ],
        out_specs=pl.BlockSpec((1, R, Cout), lambda b, i: (b, i, 0)),
        out_shape=jax.ShapeDtypeStruct((B, nq, Cout), jnp.float32),
    )(table, xyz_t, q, w1t, w2t, w3t)


def _proj_body(f_ref, fc1_ref, b1_ref, wq_ref, wk_ref, wv_ref,
               q_ref, k_ref, v_ref):
    f = f_ref[0]
    x = jnp.dot(f, fc1_ref[...], preferred_element_type=jnp.float32) + b1_ref[...]
    q_ref[0] = jnp.dot(x, wq_ref[...], preferred_element_type=jnp.float32)
    k_ref[0] = jnp.dot(x, wk_ref[...], preferred_element_type=jnp.float32)
    v_ref[0] = jnp.dot(x, wv_ref[...], preferred_element_type=jnp.float32)


def _proj_call(f, fc1t, b1, wqt, wkt, wvt):
    B, N, _ = f.shape
    d = wqt.shape[1]
    full = lambda w: pl.BlockSpec(w.shape, lambda b: (0, 0))
    out = jax.ShapeDtypeStruct((B, N, d), jnp.float32)
    return pl.pallas_call(
        _proj_body,
        grid=(B,),
        in_specs=[pl.BlockSpec((1, N, f.shape[2]), lambda b: (b, 0, 0)),
                  full(fc1t), full(b1), full(wqt), full(wkt), full(wvt)],
        out_specs=[pl.BlockSpec((1, N, d), lambda b: (b, 0, 0))] * 3,
        out_shape=[out, out, out],
    )(f, fc1t, b1, wqt, wkt, wvt)


def _attn_body(qxyz_ref, xt_ref, tab_ref, qp_ref, pre_ref,
               d1_ref, d1b_ref, d2_ref, d2b_ref,
               g1_ref, g1b_ref, g2_ref, g2b_ref,
               fc2_ref, fc2b_ref, o_ref, scr, *, K, d):
    qxyz = qxyz_ref[0]
    xt = xt_ref[0]
    tab = tab_ref[0]
    R = qxyz.shape[0]
    N = xt.shape[1]
    x2 = (xt[0:1, :] * xt[0:1, :] + xt[1:2, :] * xt[1:2, :]
          + xt[2:3, :] * xt[2:3, :])
    q2 = (qxyz[:, 0:1] * qxyz[:, 0:1] + qxyz[:, 1:2] * qxyz[:, 1:2]
          + qxyz[:, 2:3] * qxyz[:, 2:3])
    qx = jnp.dot(qxyz.astype(jnp.bfloat16), xt.astype(jnp.bfloat16),
                 preferred_element_type=jnp.float32)
    sqd = jnp.maximum(q2 - 2.0 * qx + x2, 0.0)
    iota = jax.lax.broadcasted_iota(jnp.int32, (R, N), 1).astype(jnp.float32)

    def step(j, dmat):
        mval = jnp.min(dmat, axis=1, keepdims=True)
        cur = jnp.min(jnp.where(dmat == mval, iota, float(N)), axis=1,
                      keepdims=True)
        oh = (iota == cur).astype(jnp.float32)
        val_kv = jnp.dot(oh, tab[:, :2 * d], preferred_element_type=jnp.float32)
        val_x = jnp.dot(oh, tab[:, 2 * d:], preferred_element_type=jnp.float32,
                        precision=jax.lax.Precision.HIGHEST)
        scr[pl.ds(j, 1), :, :2 * d] = val_kv[None]
        scr[pl.ds(j, 1), :, 2 * d:] = val_x[None]
        return dmat + oh * jnp.float32(1e30)

    jax.lax.fori_loop(0, K, step, sqd)

    gat = scr[...]
    kk = gat[:, :, :d]
    vv = gat[:, :, d:2 * d]
    nxyz = gat[:, :, 2 * d:]
    delta = (qxyz[None, :, :] - nxyz).reshape(K * R, 3)
    pe = jax.nn.relu(
        jnp.dot(delta, d1_ref[...], preferred_element_type=jnp.float32)
        + d1b_ref[...])
    pe = jnp.dot(pe, d2_ref[...], preferred_element_type=jnp.float32) + d2b_ref[...]
    pe3 = pe.reshape(K, R, d)
    qp = qp_ref[0]
    g = (qp[None, :, :] - kk + pe3).reshape(K * R, d)
    a = jax.nn.relu(
        jnp.dot(g, g1_ref[...], preferred_element_type=jnp.float32)
        + g1b_ref[...])
    a = jnp.dot(a, g2_ref[...], preferred_element_type=jnp.float32) + g2b_ref[...]
    a = a.reshape(K, R, d) / np.sqrt(float(d)).astype(np.float32)
    m = jnp.max(a, axis=0, keepdims=True)
    e = jnp.exp(a - m)
    a = e / jnp.sum(e, axis=0, keepdims=True)
    res = jnp.sum(a * (vv + pe3), axis=0)
    o_ref[0] = (jnp.dot(res, fc2_ref[...], preferred_element_type=jnp.float32)
                + fc2b_ref[...] + pre_ref[0])


def _attn_call(qxyz, xyz_t, tab, qp, pre, p, K, R):
    B, N, _ = qxyz.shape
    d = qp.shape[2]
    dp = pre.shape[2]
    full = lambda w: pl.BlockSpec(w.shape, lambda b, i: (0, 0))
    d1t = p['d1_w'].T
    d2t = p['d2_w'].T
    g1t = p['g1_w'].T
    g2t = p['g2_w'].T
    fc2t = p['fc2_w'].T
    row = lambda v: v[None, :]
    return pl.pallas_call(
        functools.partial(_attn_body, K=K, d=d),
        grid=(B, N // R),
        in_specs=[
            pl.BlockSpec((1, R, 3), lambda b, i: (b, i, 0)),
            pl.BlockSpec((1, 3, N), lambda b, i: (b, 0, 0)),
            pl.BlockSpec((1, N, 2 * d + 3), lambda b, i: (b, 0, 0)),
            pl.BlockSpec((1, R, d), lambda b, i: (b, i, 0)),
            pl.BlockSpec((1, R, dp), lambda b, i: (b, i, 0)),
            full(d1t), full(row(p['d1_b'])), full(d2t), full(row(p['d2_b'])),
            full(g1t), full(row(p['g1_b'])), full(g2t), full(row(p['g2_b'])),
            full(fc2t), full(row(p['fc2_b'])),
        ],
        out_specs=pl.BlockSpec((1, R, dp), lambda b, i: (b, i, 0)),
        out_shape=jax.ShapeDtypeStruct((B, N, dp), jnp.float32),
        scratch_shapes=[pltpu.VMEM((K, R, 2 * d + 3), jnp.float32)],
    )(qxyz, xyz_t, tab, qp, pre, d1t, row(p['d1_b']), d2t, row(p['d2_b']),
      g1t, row(p['g1_b']), g2t, row(p['g2_b']), fc2t, row(p['fc2_b']))


def _sa_weights(wlist):
    w1 = jnp.concatenate([wlist[0][:, 3:], wlist[0][:, :3]], axis=1)
    return w1.T, wlist[1].T, wlist[2].T


def _transformer(xyz, xyz_t, f, p, K, R):
    q, k, v = _proj_call(f, p['fc1_w'].T, p['fc1_b'][None, :],
                         p['wq'].T, p['wk'].T, p['wv'].T)
    tab = jnp.concatenate([k, v, xyz], axis=-1)
    return _attn_call(xyz, xyz_t, tab, q, f, p, K, R)


def _forward(pointcloud, params, cfg):
    pc = pointcloud.astype(jnp.float32)
    B, N, _ = pc.shape
    xyz = pc[..., :3]
    feats = pc[..., 3:]
    xyz_t = jnp.transpose(xyz, (0, 2, 1))

    np1, r1, ns1 = cfg['sa1']
    np2, r2, ns2 = cfg['sa2']

    pay1 = jnp.broadcast_to(
        jax.lax.iota(jnp.float32, N)[None, None, :], (B, 1, N))
    fps1 = _fps_call(jnp.concatenate([xyz_t, pay1], axis=1), np1)
    xyz1_t = fps1[:, 0:3, :]
    inds1f = fps1[:, 3:4, :]
    xyz1 = jnp.transpose(xyz1_t, (0, 2, 1))

    tab1 = jnp.concatenate([feats, xyz * (1.0 / r1)], axis=-1)
    f1 = _bqg_call(tab1, xyz_t, xyz1, *_sa_weights(params['sa1']),
                   r1, ns1, cfg['R_bq1'])
    f1 = _transformer(xyz1, xyz1_t, f1, params['t1'], cfg['k'], cfg['R_at1'])

    fps2 = _fps_call(jnp.concatenate([xyz1_t, inds1f], axis=1), np2)
    xyz2_t = fps2[:, 0:3, :]
    fp2f = fps2[:, 3, :]
    xyz2 = jnp.transpose(xyz2_t, (0, 2, 1))

    tab2 = jnp.concatenate([f1, xyz1 * (1.0 / r2)], axis=-1)
    f2 = _bqg_call(tab2, xyz1_t, xyz2, *_sa_weights(params['sa2']),
                   r2, ns2, cfg['R_bq2'])
    f2 = _transformer(xyz2, xyz2_t, f2, params['t2'], cfg['k'], cfg['R_at2'])

    return (jnp.transpose(f2, (0, 2, 1)), xyz2, fp2f.astype(jnp.int32))


_CFG = {
    'sa1': (2048, 0.04, 64),
    'sa2': (1024, 0.1, 32),
    'k': 16,
    'R_bq1': 256,
    'R_bq2': 256,
    'R_at1': 128,
    'R_at2': 128,
}


def kernel(pointcloud, params):
    return _forward(pointcloud, params, _CFG)


# speedup vs baseline: 1.2550x; 1.0732x over previous
import functools
import numpy as np
import jax
import jax.numpy as jnp
from jax.experimental import pallas as pl
from jax.experimental.pallas import tpu as pltpu

_BN = 1.0 / np.sqrt(1.0 + 1e-5)


def _fps_body(x_ref, o_ref, *, npoint):
    x = x_ref[...]
    B, _, N = x.shape
    iota = jax.lax.broadcasted_iota(jnp.int32, (B, N), 1).astype(jnp.float32)

    def step(i, carry):
        dists, far = carry
        oh = (iota == far).astype(x.dtype)
        ct = jnp.sum(x * oh[:, None, :], axis=2)
        dx = x[:, 0, :] - ct[:, 0:1]
        dy = x[:, 1, :] - ct[:, 1:2]
        dz = x[:, 2, :] - ct[:, 2:3]
        d = (dx * dx + dy * dy) + dz * dz
        dists = jnp.minimum(dists, d)
        m = jnp.max(dists, axis=1, keepdims=True)
        nxt = jnp.min(jnp.where(dists == m, iota, float(N)), axis=1,
                      keepdims=True)
        o_ref[pl.ds(i, 1), :, :] = ct[None]
        return dists, nxt

    jax.lax.fori_loop(
        0, npoint, step,
        (jnp.full((B, N), 1e10, x.dtype), jnp.zeros((B, 1), x.dtype)))


def _fps_call(xpay, npoint):
    B = xpay.shape[0]
    out = pl.pallas_call(
        functools.partial(_fps_body, npoint=npoint),
        out_shape=jax.ShapeDtypeStruct((npoint, B, 4), jnp.float32),
    )(xpay)
    return jnp.transpose(out, (1, 2, 0))


def _bqg_body(tab_ref, xt_ref, q_ref, w1_ref, w2_ref, w3_ref, o_ref,
              *, r2, inv_r, ns, C, ch):
    tab = tab_ref[0]
    xt = xt_ref[0]
    q = q_ref[0]
    R = q.shape[0]
    N = xt.shape[1]
    x2 = (xt[0:1, :] * xt[0:1, :] + xt[1:2, :] * xt[1:2, :]
          + xt[2:3, :] * xt[2:3, :])
    q2 = (q[:, 0:1] * q[:, 0:1] + q[:, 1:2] * q[:, 1:2]
          + q[:, 2:3] * q[:, 2:3])
    qx = jnp.dot(q.astype(jnp.bfloat16), xt.astype(jnp.bfloat16),
                 preferred_element_type=jnp.float32)
    sqd = jnp.maximum(q2 - 2.0 * qx + x2, 0.0)
    mask = (sqd < r2).astype(jnp.float32)
    iota = jax.lax.broadcasted_iota(jnp.int32, (R, ch), 1).astype(jnp.float32)
    shift = jnp.concatenate(
        [jnp.zeros((R, C), jnp.float32), q * inv_r], axis=1)
    Cout = w3_ref.shape[1]

    def extract_mlp(oh, tab_c):
        v = jnp.dot(oh, tab_c, preferred_element_type=jnp.float32,
                    precision=jax.lax.Precision.HIGHEST) - shift
        v = jax.nn.relu(jnp.dot(v, w1_ref[...],
                                preferred_element_type=jnp.float32) * _BN)
        v = jax.nn.relu(jnp.dot(v, w2_ref[...],
                                preferred_element_type=jnp.float32) * _BN)
        return jax.nn.relu(jnp.dot(v, w3_ref[...],
                                   preferred_element_type=jnp.float32) * _BN)

    acc = jnp.full((R, Cout), -1e30, jnp.float32)
    cnt = jnp.zeros((R, 1), jnp.float32)
    for c in range(N // ch):
        m0 = mask[:, c * ch:(c + 1) * ch]
        tab_c = tab[c * ch:(c + 1) * ch, :]
        limit_c = jnp.max(jnp.sum(m0, axis=1)).astype(jnp.int32)

        def cstep(j, carry, tab_c=tab_c):
            m, cnt, acc = carry
            cur = jnp.min(jnp.where(m > 0.0, iota, float(ch)), axis=1,
                          keepdims=True)
            ok = jnp.logical_and(cur < ch, cnt < float(ns))
            oh = (iota == cur).astype(jnp.float32)
            val = extract_mlp(oh, tab_c)
            acc = jnp.where(ok, jnp.maximum(acc, val), acc)
            cnt = cnt + ok.astype(jnp.float32)
            return m - oh, cnt, acc

        _, cnt, acc = jax.lax.fori_loop(0, limit_c, cstep, (m0, cnt, acc))

    oh_last = (iota == float(ch - 1)).astype(jnp.float32)
    padv = extract_mlp(oh_last, tab[N - ch:N, :])
    o_ref[0] = jnp.where(cnt > 0.0, acc, padv)


def _bqg_call(table, xyz_t, q, w1t, w2t, w3t, radius, ns, R):
    B, N, C3 = table.shape
    nq = q.shape[1]
    Cout = w3t.shape[1]
    full = lambda w: pl.BlockSpec(w.shape, lambda b, i: (0, 0))
    return pl.pallas_call(
        functools.partial(_bqg_body, r2=radius * radius, inv_r=1.0 / radius,
                          ns=ns, C=C3 - 3, ch=min(512, N)),
        grid=(B, nq // R),
        in_specs=[
            pl.BlockSpec((1, N, C3), lambda b, i: (b, 0, 0)),
            pl.BlockSpec((1, 3, N), lambda b, i: (b, 0, 0)),
            pl.BlockSpec((1, R, 3), lambda b, i: (b, i, 0)),
            full(w1t), full(w2t), full(w3t),
        ],
        out_specs=pl.BlockSpec((1, R, Cout), lambda b, i: (b, i, 0)),
        out_shape=jax.ShapeDtypeStruct((B, nq, Cout), jnp.float32),
    )(table, xyz_t, q, w1t, w2t, w3t)


def _proj_body(f_ref, fc1_ref, b1_ref, wq_ref, wk_ref, wv_ref,
               q_ref, k_ref, v_ref):
    f = f_ref[0]
    x = jnp.dot(f, fc1_ref[...], preferred_element_type=jnp.float32) + b1_ref[...]
    q_ref[0] = jnp.dot(x, wq_ref[...], preferred_element_type=jnp.float32)
    k_ref[0] = jnp.dot(x, wk_ref[...], preferred_element_type=jnp.float32)
    v_ref[0] = jnp.dot(x, wv_ref[...], preferred_element_type=jnp.float32)


def _proj_call(f, fc1t, b1, wqt, wkt, wvt):
    B, N, _ = f.shape
    d = wqt.shape[1]
    full = lambda w: pl.BlockSpec(w.shape, lambda b: (0, 0))
    out = jax.ShapeDtypeStruct((B, N, d), jnp.float32)
    return pl.pallas_call(
        _proj_body,
        grid=(B,),
        in_specs=[pl.BlockSpec((1, N, f.shape[2]), lambda b: (b, 0, 0)),
                  full(fc1t), full(b1), full(wqt), full(wkt), full(wvt)],
        out_specs=[pl.BlockSpec((1, N, d), lambda b: (b, 0, 0))] * 3,
        out_shape=[out, out, out],
    )(f, fc1t, b1, wqt, wkt, wvt)


def _attn_body(qxyz_ref, xt_ref, tab_ref, qp_ref, pre_ref,
               d1_ref, d1b_ref, d2_ref, d2b_ref,
               g1_ref, g1b_ref, g2_ref, g2b_ref,
               fc2_ref, fc2b_ref, o_ref, scr, *, K, d):
    qxyz = qxyz_ref[0]
    xt = xt_ref[0]
    tab = tab_ref[0]
    R = qxyz.shape[0]
    N = xt.shape[1]
    x2 = (xt[0:1, :] * xt[0:1, :] + xt[1:2, :] * xt[1:2, :]
          + xt[2:3, :] * xt[2:3, :])
    q2 = (qxyz[:, 0:1] * qxyz[:, 0:1] + qxyz[:, 1:2] * qxyz[:, 1:2]
          + qxyz[:, 2:3] * qxyz[:, 2:3])
    qx = jnp.dot(qxyz.astype(jnp.bfloat16), xt.astype(jnp.bfloat16),
                 preferred_element_type=jnp.float32)
    sqd = jnp.maximum(q2 - 2.0 * qx + x2, 0.0)
    iota = jax.lax.broadcasted_iota(jnp.int32, (R, N), 1).astype(jnp.float32)

    def step(j, dmat):
        mval = jnp.min(dmat, axis=1, keepdims=True)
        cur = jnp.min(jnp.where(dmat == mval, iota, float(N)), axis=1,
                      keepdims=True)
        oh = (iota == cur).astype(jnp.float32)
        val_kv = jnp.dot(oh, tab[:, :2 * d], preferred_element_type=jnp.float32)
        val_x = jnp.dot(oh, tab[:, 2 * d:], preferred_element_type=jnp.float32,
                        precision=jax.lax.Precision.HIGHEST)
        scr[pl.ds(j, 1), :, :2 * d] = val_kv[None]
        scr[pl.ds(j, 1), :, 2 * d:] = val_x[None]
        return dmat + oh * jnp.float32(1e30)

    jax.lax.fori_loop(0, K, step, sqd)

    gat = scr[...]
    kk = gat[:, :, :d]
    vv = gat[:, :, d:2 * d]
    nxyz = gat[:, :, 2 * d:]
    delta = (qxyz[None, :, :] - nxyz).reshape(K * R, 3)
    pe = jax.nn.relu(
        jnp.dot(delta, d1_ref[...], preferred_element_type=jnp.float32)
        + d1b_ref[...])
    pe = jnp.dot(pe, d2_ref[...], preferred_element_type=jnp.float32) + d2b_ref[...]
    pe3 = pe.reshape(K, R, d)
    qp = qp_ref[0]
    g = (qp[None, :, :] - kk + pe3).reshape(K * R, d)
    a = jax.nn.relu(
        jnp.dot(g, g1_ref[...], preferred_element_type=jnp.float32)
        + g1b_ref[...])
    a = jnp.dot(a, g2_ref[...], preferred_element_type=jnp.float32) + g2b_ref[...]
    a = a.reshape(K, R, d) / np.sqrt(float(d)).astype(np.float32)
    m = jnp.max(a, axis=0, keepdims=True)
    e = jnp.exp(a - m)
    a = e / jnp.sum(e, axis=0, keepdims=True)
    res = jnp.sum(a * (vv + pe3), axis=0)
    o_ref[0] = (jnp.dot(res, fc2_ref[...], preferred_element_type=jnp.float32)
                + fc2b_ref[...] + pre_ref[0])


def _attn_call(qxyz, xyz_t, tab, qp, pre, p, K, R):
    B, N, _ = qxyz.shape
    d = qp.shape[2]
    dp = pre.shape[2]
    full = lambda w: pl.BlockSpec(w.shape, lambda b, i: (0, 0))
    d1t = p['d1_w'].T
    d2t = p['d2_w'].T
    g1t = p['g1_w'].T
    g2t = p['g2_w'].T
    fc2t = p['fc2_w'].T
    row = lambda v: v[None, :]
    return pl.pallas_call(
        functools.partial(_attn_body, K=K, d=d),
        grid=(B, N // R),
        in_specs=[
            pl.BlockSpec((1, R, 3), lambda b, i: (b, i, 0)),
            pl.BlockSpec((1, 3, N), lambda b, i: (b, 0, 0)),
            pl.BlockSpec((1, N, 2 * d + 3), lambda b, i: (b, 0, 0)),
            pl.BlockSpec((1, R, d), lambda b, i: (b, i, 0)),
            pl.BlockSpec((1, R, dp), lambda b, i: (b, i, 0)),
            full(d1t), full(row(p['d1_b'])), full(d2t), full(row(p['d2_b'])),
            full(g1t), full(row(p['g1_b'])), full(g2t), full(row(p['g2_b'])),
            full(fc2t), full(row(p['fc2_b'])),
        ],
        out_specs=pl.BlockSpec((1, R, dp), lambda b, i: (b, i, 0)),
        out_shape=jax.ShapeDtypeStruct((B, N, dp), jnp.float32),
        scratch_shapes=[pltpu.VMEM((K, R, 2 * d + 3), jnp.float32)],
    )(qxyz, xyz_t, tab, qp, pre, d1t, row(p['d1_b']), d2t, row(p['d2_b']),
      g1t, row(p['g1_b']), g2t, row(p['g2_b']), fc2t, row(p['fc2_b']))


def _sa_weights(wlist):
    w1 = jnp.concatenate([wlist[0][:, 3:], wlist[0][:, :3]], axis=1)
    return w1.T, wlist[1].T, wlist[2].T


def _transformer(xyz, xyz_t, f, p, K, R):
    q, k, v = _proj_call(f, p['fc1_w'].T, p['fc1_b'][None, :],
                         p['wq'].T, p['wk'].T, p['wv'].T)
    tab = jnp.concatenate([k, v, xyz], axis=-1)
    return _attn_call(xyz, xyz_t, tab, q, f, p, K, R)


def _forward(pointcloud, params, cfg):
    pc = pointcloud.astype(jnp.float32)
    B, N, _ = pc.shape
    xyz = pc[..., :3]
    feats = pc[..., 3:]
    xyz_t = jnp.transpose(xyz, (0, 2, 1))

    np1, r1, ns1 = cfg['sa1']
    np2, r2, ns2 = cfg['sa2']

    pay1 = jnp.broadcast_to(
        jax.lax.iota(jnp.float32, N)[None, None, :], (B, 1, N))
    fps1 = _fps_call(jnp.concatenate([xyz_t, pay1], axis=1), np1)
    xyz1_t = fps1[:, 0:3, :]
    inds1f = fps1[:, 3:4, :]
    xyz1 = jnp.transpose(xyz1_t, (0, 2, 1))

    tab1 = jnp.concatenate([feats, xyz * (1.0 / r1)], axis=-1)
    f1 = _bqg_call(tab1, xyz_t, xyz1, *_sa_weights(params['sa1']),
                   r1, ns1, cfg['R_bq1'])
    f1 = _transformer(xyz1, xyz1_t, f1, params['t1'], cfg['k'], cfg['R_at1'])

    fps2 = _fps_call(jnp.concatenate([xyz1_t, inds1f], axis=1), np2)
    xyz2_t = fps2[:, 0:3, :]
    fp2f = fps2[:, 3, :]
    xyz2 = jnp.transpose(xyz2_t, (0, 2, 1))

    tab2 = jnp.concatenate([f1, xyz1 * (1.0 / r2)], axis=-1)
    f2 = _bqg_call(tab2, xyz1_t, xyz2, *_sa_weights(params['sa2']),
                   r2, ns2, cfg['R_bq2'])
    f2 = _transformer(xyz2, xyz2_t, f2, params['t2'], cfg['k'], cfg['R_at2'])

    return (jnp.transpose(f2, (0, 2, 1)), xyz2, fp2f.astype(jnp.int32))


_CFG = {
    'sa1': (2048, 0.04, 64),
    'sa2': (1024, 0.1, 32),
    'k': 16,
    'R_bq1': 512,
    'R_bq2': 512,
    'R_at1': 256,
    'R_at2': 128,
}


def kernel(pointcloud, params):
    return _forward(pointcloud, params, _CFG)
